# baseline (speedup 1.0000x reference)
import functools

import jax
import jax.numpy as jnp
from jax import lax
from jax.experimental import pallas as pl
from jax.experimental.pallas import tpu as pltpu
from jax.experimental.pallas import tpu_sc as plsc

_B = 4096
_H = 200
_D = 64
_N = _B * _H
_NC = 2
_NS = 16
_NW = _NC * _NS
_PER_W = _N // _NW
_IROW = 128
_KROWS = 8
_CHUNK = _IROW * _KROWS
_NCHUNK = _PER_W // _CHUNK


def _make_gather():
    mesh = plsc.VectorSubcoreMesh(core_axis_name="c", subcore_axis_name="s")

    @functools.partial(
        pl.kernel,
        mesh=mesh,
        out_type=jax.ShapeDtypeStruct((_N, _D), jnp.float32),
        scratch_types=[
            pltpu.VMEM((_KROWS, _IROW), jnp.int32),
            pltpu.VMEM((_CHUNK, _D), jnp.float32),
            pltpu.SemaphoreType.DMA,
        ],
        compiler_params=pltpu.CompilerParams(use_tc_tiling_on_sc=False),
    )
    def gather_kernel(table_hbm, idx_hbm, out_hbm, idx_v, rows_v, sem):
        wid = lax.axis_index("s") * _NC + lax.axis_index("c")
        base = wid * _PER_W

        def body(g, carry):
            off = base + g * _CHUNK
            irow_off = pl.multiple_of(off // _IROW, 8)
            pltpu.sync_copy(
                idx_hbm.at[pl.ds(irow_off, _KROWS)], idx_v
            )
            for j in range(_KROWS):
                pltpu.async_copy(
                    table_hbm.at[idx_v.at[j]],
                    rows_v.at[pl.ds(j * _IROW, _IROW)],
                    sem,
                )
            for j in range(_KROWS):
                pltpu.make_async_copy(
                    table_hbm.at[idx_v.at[j]],
                    rows_v.at[pl.ds(j * _IROW, _IROW)],
                    sem,
                ).wait()
            pltpu.sync_copy(rows_v, out_hbm.at[pl.ds(off, _CHUNK)])
            return carry

        lax.fori_loop(0, _NCHUNK, body, 0)

    return gather_kernel


_gather = _make_gather()


def kernel(input_, weight):
    idx = input_.reshape(_N // _IROW, _IROW).astype(jnp.int32)
    out = _gather(weight, idx)
    return out.reshape(_B, _H, _D)


# speedup vs baseline: 1.0170x; 1.0170x over previous
import functools

import jax
import jax.numpy as jnp
from jax import lax
from jax.experimental import pallas as pl
from jax.experimental.pallas import tpu as pltpu
from jax.experimental.pallas import tpu_sc as plsc

_B = 4096
_H = 200
_D = 64
_N = _B * _H
_NC = 2
_NS = 16
_NW = _NC * _NS
_PER_W = _N // _NW
_IROW = 128
_KROWS = 4
_CHUNK = _IROW * _KROWS
_NCHUNK = _PER_W // _CHUNK


def _make_gather():
    mesh = plsc.VectorSubcoreMesh(core_axis_name="c", subcore_axis_name="s")

    @functools.partial(
        pl.kernel,
        mesh=mesh,
        out_type=jax.ShapeDtypeStruct((_N, _D), jnp.float32),
        scratch_types=[
            pltpu.VMEM((_PER_W,), jnp.int32),
            pltpu.VMEM((2, _CHUNK, _D), jnp.float32),
            pltpu.SemaphoreType.DMA,
            pltpu.SemaphoreType.DMA,
            pltpu.SemaphoreType.DMA,
            pltpu.SemaphoreType.DMA,
        ],
        compiler_params=pltpu.CompilerParams(use_tc_tiling_on_sc=False),
    )
    def gather_kernel(table_hbm, idx_hbm, out_hbm, idx_v, rows_v, sg0, sg1,
                      ss0, ss1):
        wid = lax.axis_index("s") * _NC + lax.axis_index("c")
        base = wid * _PER_W
        sg = (sg0, sg1)
        ss = (ss0, ss1)

        pltpu.sync_copy(idx_hbm.at[pl.ds(base, _PER_W)], idx_v)

        def start_gather(c, b):
            for j in range(_KROWS):
                pltpu.async_copy(
                    table_hbm.at[idx_v.at[pl.ds(c * _CHUNK + j * _IROW,
                                                _IROW)]],
                    rows_v.at[b, pl.ds(j * _IROW, _IROW)],
                    sg[b],
                )

        def wait_gather(c, b):
            for j in range(_KROWS):
                pltpu.make_async_copy(
                    table_hbm.at[idx_v.at[pl.ds(c * _CHUNK + j * _IROW,
                                                _IROW)]],
                    rows_v.at[b, pl.ds(j * _IROW, _IROW)],
                    sg[b],
                ).wait()

        def start_store(c, b):
            pltpu.async_copy(
                rows_v.at[b], out_hbm.at[pl.ds(base + c * _CHUNK, _CHUNK)],
                ss[b],
            )

        def wait_store(c, b):
            pltpu.make_async_copy(
                rows_v.at[b], out_hbm.at[pl.ds(base + c * _CHUNK, _CHUNK)],
                ss[b],
            ).wait()

        start_gather(0, 0)
        start_gather(1, 1)

        def body(t, carry):
            for b in range(2):
                c = t * 2 + b
                wait_gather(c, b)
                start_store(c, b)
                wait_store(c, b)
                start_gather(c + 2, b)
            return carry

        lax.fori_loop(0, (_NCHUNK - 2) // 2, body, 0)

        for b in range(2):
            c = _NCHUNK - 2 + b
            wait_gather(c, b)
            start_store(c, b)
        for b in range(2):
            wait_store(_NCHUNK - 2 + b, b)

    return gather_kernel


_gather = _make_gather()


def kernel(input_, weight):
    idx = input_.reshape(_N).astype(jnp.int32)
    out = _gather(weight, idx)
    return out.reshape(_B, _H, _D)
